# baseline (speedup 1.0000x reference)
import jax
import jax.numpy as jnp
from jax.experimental import pallas as pl
from jax.experimental.pallas import tpu as pltpu

_EPS = 1e-5
_B, _R, _CF, _FM = 8, 64, 512, 8
_NROWS = _B * _R
_PK = 64
_NT = 1024
_NSTEPS = 4096 // _NT


def _fold_bn(w_oihw, bn):
    g, b, m, v = bn
    s = g / jnp.sqrt(v + _EPS)
    w = jnp.transpose(w_oihw * s[:, None, None, None], (2, 3, 1, 0))
    return w, (b - m * s)


def _convf(x, w, bias, stride, pad):
    y = jax.lax.conv_general_dilated(
        x, w, (stride, stride), [(pad, pad), (pad, pad)],
        dimension_numbers=('NHWC', 'HWIO', 'NHWC'))
    return y + bias


def _backbone(x_nhwc, params):
    w, b = _fold_bn(params['conv1'], params['bn1'])
    x = jax.nn.relu(_convf(x_nhwc, w, b, 2, 3))
    x = jax.lax.reduce_window(x, -jnp.inf, jax.lax.max, (1, 3, 3, 1),
                              (1, 2, 2, 1), [(0, 0), (1, 1), (1, 1), (0, 0)])
    for li in range(4):
        for bi in range(len(params['layers'][li])):
            p = params['layers'][li][bi]
            stride = 2 if (li > 0 and bi == 0) else 1
            if 'down' in p:
                wd, bd = _fold_bn(p['down'][0], p['down'][1])
                identity = _convf(x, wd, bd, stride, 0)
            else:
                identity = x
            w1, b1 = _fold_bn(p['conv1'], p['bn1'])
            out = jax.nn.relu(_convf(x, w1, b1, stride, 1))
            w2, b2 = _fold_bn(p['conv2'], p['bn2'])
            out = _convf(out, w2, b2, 1, 1)
            x = jax.nn.relu(out + identity)
    return x


def _pool_fc6_kernel(fm_ref, ssw_ref, w6_ref, b6_ref, out_ref):
    fmf = fm_ref[0]
    hm = jnp.maximum(fmf, jnp.roll(fmf, -1, axis=0))
    pmat = jnp.maximum(hm, jnp.roll(hm, -8, axis=0))

    ssw = ssw_ref[0]
    base = ssw[:, 0:1] * _FM + ssw[:, 1:2]
    lanes = jax.lax.broadcasted_iota(jnp.int32, (_R, _PK), 1)
    pooled = []
    for i in range(2):
        for j in range(2):
            idx = base + (2 * i) * _FM + 2 * j
            onehot = (lanes == idx).astype(jnp.float32)
            pooled.append(jnp.dot(onehot, pmat,
                                  preferred_element_type=jnp.float32))
    flat = jnp.concatenate(pooled, axis=1)
    h = jnp.dot(flat, w6_ref[...], preferred_element_type=jnp.float32)
    out_ref[...] = jnp.maximum(h + b6_ref[...], 0.0)


def _pool_fc6(feats_rows, ssw, w6p, b6):
    return pl.pallas_call(
        _pool_fc6_kernel,
        grid=(_B,),
        in_specs=[
            pl.BlockSpec((1, _FM * _FM, _CF), lambda b: (b, 0, 0)),
            pl.BlockSpec((1, _R, 4), lambda b: (b, 0, 0)),
            pl.BlockSpec((2048, 4096), lambda b: (0, 0)),
            pl.BlockSpec((1, 4096), lambda b: (0, 0)),
        ],
        out_specs=pl.BlockSpec((_R, 4096), lambda b: (b, 0)),
        out_shape=jax.ShapeDtypeStruct((_NROWS, 4096), jnp.float32),
        compiler_params=pltpu.CompilerParams(
            dimension_semantics=("arbitrary",)),
    )(feats_rows, ssw, w6p, b6)


def _head_kernel(h1_ref, w7_ref, b7_ref, w8c_ref, w8d_ref, b8c_ref, b8d_ref,
                 out_ref, sd_ref, sc_ref, xc_acc, xd_acc):
    n = pl.program_id(0)
    h2 = jnp.dot(h1_ref[...], w7_ref[...], preferred_element_type=jnp.float32)
    h2 = jnp.maximum(h2 + b7_ref[...], 0.0)
    xc = jnp.dot(h2, w8c_ref[...], preferred_element_type=jnp.float32)
    xd = jnp.dot(h2, w8d_ref[...], preferred_element_type=jnp.float32)

    @pl.when(n == 0)
    def _():
        xc_acc[...] = xc
        xd_acc[...] = xd

    @pl.when(n > 0)
    def _():
        xc_acc[...] += xc
        xd_acc[...] += xd

    @pl.when(n == _NSTEPS - 1)
    def _():
        xcf = jnp.maximum(xc_acc[...] + b8c_ref[...], 0.0).reshape(_B, _R, 2)
        xdf = jnp.maximum(xd_acc[...] + b8d_ref[...], 0.0).reshape(_B, _R, 2)
        ec = jnp.exp(xcf - jnp.max(xcf, axis=2, keepdims=True))
        sc = ec / jnp.sum(ec, axis=2, keepdims=True)
        ed = jnp.exp(xdf - jnp.max(xdf, axis=1, keepdims=True))
        sd = ed / jnp.sum(ed, axis=1, keepdims=True)
        out_ref[...] = jnp.sum(sc * sd, axis=1)
        sd_ref[...] = sd
        sc_ref[...] = sc


def _head(h1, w7t, b7, w8ct, w8dt, b8c, b8d):
    return pl.pallas_call(
        _head_kernel,
        grid=(_NSTEPS,),
        in_specs=[
            pl.BlockSpec((_NROWS, 4096), lambda n: (0, 0)),
            pl.BlockSpec((4096, _NT), lambda n: (0, n)),
            pl.BlockSpec((1, _NT), lambda n: (0, n)),
            pl.BlockSpec((_NT, 2), lambda n: (n, 0)),
            pl.BlockSpec((_NT, 2), lambda n: (n, 0)),
            pl.BlockSpec((1, 2), lambda n: (0, 0)),
            pl.BlockSpec((1, 2), lambda n: (0, 0)),
        ],
        out_specs=[
            pl.BlockSpec((_B, 2), lambda n: (0, 0)),
            pl.BlockSpec((_B, _R, 2), lambda n: (0, 0, 0)),
            pl.BlockSpec((_B, _R, 2), lambda n: (0, 0, 0)),
        ],
        out_shape=[
            jax.ShapeDtypeStruct((_B, 2), jnp.float32),
            jax.ShapeDtypeStruct((_B, _R, 2), jnp.float32),
            jax.ShapeDtypeStruct((_B, _R, 2), jnp.float32),
        ],
        scratch_shapes=[
            pltpu.VMEM((_NROWS, 2), jnp.float32),
            pltpu.VMEM((_NROWS, 2), jnp.float32),
        ],
        compiler_params=pltpu.CompilerParams(
            dimension_semantics=("arbitrary",)),
    )(h1, w7t, b7, w8ct, w8dt, b8c, b8d)


def kernel(x, ssw_get, params):
    feats = _backbone(jnp.transpose(x, (0, 2, 3, 1)), params)
    feats_rows = feats.reshape(_B, _FM * _FM, _CF)

    w6, b6 = params['fc6']
    w6e = w6[:, :2048] + w6[:, 2048:]
    w6p = jnp.transpose(w6e.reshape(4096, _CF, 4), (2, 1, 0)).reshape(2048, 4096)

    w7, b7 = params['fc7']
    w8c, b8c = params['fc8c']
    w8d, b8d = params['fc8d']

    h1 = _pool_fc6(feats_rows, ssw_get, w6p, b6.reshape(1, 4096))
    out, sd, sc = _head(h1, w7.T, b7.reshape(1, 4096),
                        w8c.T, w8d.T, b8c.reshape(1, 2), b8d.reshape(1, 2))
    return out, sd, sc
